# Initial kernel scaffold; baseline (speedup 1.0000x reference)
import jax
import jax.numpy as jnp
from jax.experimental import pallas as pl


def kernel(x, edge_index, W1, b1, bn_w, bn_b, W2, b2, gamma):
    raise NotImplementedError("write your pallas kernel here")


# speedup vs baseline: 5.0934x; 5.0934x over previous
import functools

import jax
import jax.numpy as jnp
from jax import lax
from jax.experimental import pallas as pl
from jax.experimental.pallas import tpu as pltpu
from jax.experimental.pallas import tpu_sc as plsc

BN_EPS = 1e-5
NC = 2
NS = 16
NW = NC * NS
B = 128


NBUF = 6


def _edge_body(n_chunks, npad, y2, row3a, row3b, col3, zblk, p_out,
               rowia, rowib, coli, buf, gsems, ssems, acc):
    cid = lax.axis_index("c")
    sid = lax.axis_index("s")
    wid = cid * NS + sid
    rt = npad // NS

    pltpu.sync_copy(row3a.at[wid], rowia)
    pltpu.sync_copy(row3b.at[wid], rowib)
    pltpu.sync_copy(col3.at[wid], coli)

    for h, rowi in ((0, rowia), (1, rowib)):
        plsc.subcore_barrier()
        pltpu.sync_copy(zblk, acc.at[pl.ds(sid * rt, rt)])
        plsc.subcore_barrier()

        gd = {}
        sd = {}
        for b in range(min(3, n_chunks)):
            gd[b] = pltpu.async_copy(y2.at[rowi.at[b]], buf.at[b], gsems.at[b])
        for j in range(n_chunks):
            b = j % NBUF
            pj = j + 3
            if pj < n_chunks:
                pb = pj % NBUF
                if pb in sd:
                    sd.pop(pb).wait()
                gd[pb] = pltpu.async_copy(
                    y2.at[rowi.at[pj]], buf.at[pb], gsems.at[pb])
            gd.pop(b).wait()
            sd[b] = pltpu.async_copy(
                buf.at[b], acc.at[coli.at[j]], ssems.at[b], add=True)
        for b in list(sd):
            sd.pop(b).wait()

        plsc.subcore_barrier()
        pltpu.sync_copy(
            acc.at[pl.ds(sid * rt, rt)],
            p_out.at[pl.ds(cid * 2 * npad + h * npad + sid * rt, rt)])


def _mlp_body(gamma_ref, x_ref, w1_ref, b1_ref, bnw_ref, bnb_ref, w2_ref,
              b2_ref, degp_ref, y0_ref, g0_ref, dinv16_ref, dsq16_ref):
    h = lax.dot_general(x_ref[...], w1_ref[...], (((1,), (1,)), ((), ())),
                        preferred_element_type=jnp.float32)
    scale = bnw_ref[...] * lax.rsqrt(jnp.float32(1.0 + BN_EPS))
    h = h * scale + b1_ref[...] * scale + bnb_ref[...]
    h = jnp.maximum(h, 0.0)
    h = lax.dot_general(h, w2_ref[...], (((1,), (1,)), ((), ())),
                        preferred_element_type=jnp.float32) + b2_ref[...]
    d = jnp.maximum(degp_ref[:, 0:1] + degp_ref[:, 1:2], 1.0)
    dis = lax.rsqrt(d)
    y0 = h * dis
    y0_ref[...] = y0
    g0_ref[...] = y0 * gamma_ref[0]
    ones16 = jnp.ones((1, 16), jnp.float32)
    dinv16_ref[...] = (dis * dis) * ones16
    dsq16_ref[...] = (d * dis) * ones16


def _combine_body(gk_ref, p0_ref, p1_ref, dinv16_ref, g_ref, y_ref, gout_ref):
    y = (p0_ref[...] + p1_ref[...]) * dinv16_ref[:, 0:1]
    y_ref[...] = y
    gout_ref[...] = g_ref[...] + y * gk_ref[0]


def _final_body(hc, dsq16_ref, ga_ref, gb_ref, out_ref):
    d = dsq16_ref[:, 0:1]
    out_ref[:, :hc] = ga_ref[...] * d
    out_ref[:, hc:] = gb_ref[...] * d


def kernel(x, edge_index, W1, b1, bn_w, bn_b, W2, b2, gamma):
    n, in_ch = x.shape
    e = edge_index.shape[1]
    kk = gamma.shape[0] - 1
    out_ch = W2.shape[0]

    n_chunks = -(-e // (NW * B))
    e_pad = NW * B * n_chunks
    npad = -(-(n + 1) // 256) * 256
    rt = npad // NS

    row = edge_index[0]
    col = edge_index[1]
    pad_idx = jnp.full((e_pad - e,), n, jnp.int32)
    row3 = jnp.concatenate([row, pad_idx]).reshape(NW, n_chunks, B)
    col3 = jnp.concatenate([col, pad_idx]).reshape(NW, n_chunks, B)
    x_p = jnp.pad(x, ((0, npad - n), (0, 0)))
    zblk = jnp.zeros((rt, out_ch // 2), jnp.float32)

    mesh = plsc.VectorSubcoreMesh(core_axis_name="c", subcore_axis_name="s",
                                  num_cores=NC, num_subcores=NS)
    sc_params = pltpu.CompilerParams(needs_layout_passes=False,
                                     use_tc_tiling_on_sc=False)

    hc = out_ch // 2
    edge_call = pl.kernel(
        functools.partial(_edge_body, n_chunks, npad),
        out_type=jax.ShapeDtypeStruct((NC * 2 * npad, hc), jnp.float32),
        mesh=mesh,
        scratch_types=[
            pltpu.VMEM((n_chunks, B), jnp.int32),
            pltpu.VMEM((n_chunks, B), jnp.int32),
            pltpu.VMEM((n_chunks, B), jnp.int32),
            pltpu.VMEM((NBUF, B, hc), jnp.float32),
            pltpu.SemaphoreType.DMA((NBUF,)),
            pltpu.SemaphoreType.DMA((NBUF,)),
            pltpu.VMEM_SHARED((npad, hc), jnp.float32),
        ],
        compiler_params=sc_params,
    )

    ones2 = jnp.ones((2 * npad, hc), jnp.float32)
    p_deg = edge_call(ones2, row3, row3, row3, zblk)
    degp = jnp.stack([p_deg[:npad, 0], p_deg[2 * npad:3 * npad, 0]], axis=1)

    grid = npad // rt

    mlp_call = pl.pallas_call(
        _mlp_body,
        grid=(grid,),
        in_specs=[
            pl.BlockSpec(memory_space=pltpu.SMEM),
            pl.BlockSpec((rt, in_ch), lambda i: (i, 0)),
            pl.BlockSpec(W1.shape, lambda i: (0, 0)),
            pl.BlockSpec((1, in_ch), lambda i: (0, 0)),
            pl.BlockSpec((1, in_ch), lambda i: (0, 0)),
            pl.BlockSpec((1, in_ch), lambda i: (0, 0)),
            pl.BlockSpec(W2.shape, lambda i: (0, 0)),
            pl.BlockSpec((1, out_ch), lambda i: (0, 0)),
            pl.BlockSpec((rt, NC), lambda i: (i, 0)),
        ],
        out_specs=[
            pl.BlockSpec((rt, out_ch), lambda i: (i, 0)),
            pl.BlockSpec((rt, out_ch), lambda i: (i, 0)),
            pl.BlockSpec((rt, 16), lambda i: (i, 0)),
            pl.BlockSpec((rt, 16), lambda i: (i, 0)),
        ],
        out_shape=[
            jax.ShapeDtypeStruct((npad, out_ch), jnp.float32),
            jax.ShapeDtypeStruct((npad, out_ch), jnp.float32),
            jax.ShapeDtypeStruct((npad, 16), jnp.float32),
            jax.ShapeDtypeStruct((npad, 16), jnp.float32),
        ],
    )
    y0, g0, dinv16, dsq16 = mlp_call(
        gamma, x_p, W1, b1.reshape(1, -1), bn_w.reshape(1, -1),
        bn_b.reshape(1, -1), W2, b2.reshape(1, -1), degp)

    y2_0 = jnp.concatenate([y0[:, :hc], y0[:, hc:]], axis=0)
    g2_0 = jnp.concatenate([g0[:, :hc], g0[:, hc:]], axis=0)
    row3b = row3 + npad

    grid2 = 2 * grid
    combine_call = pl.pallas_call(
        _combine_body,
        grid=(grid2,),
        in_specs=[
            pl.BlockSpec(memory_space=pltpu.SMEM),
            pl.BlockSpec((rt, hc), lambda i: (i, 0)),
            pl.BlockSpec((rt, hc), lambda i: (i + 2 * grid, 0)),
            pl.BlockSpec((rt, 16), lambda i: (i % grid, 0)),
            pl.BlockSpec((rt, hc), lambda i: (i, 0)),
        ],
        out_specs=[pl.BlockSpec((rt, hc), lambda i: (i, 0)),
                   pl.BlockSpec((rt, hc), lambda i: (i, 0))],
        out_shape=[jax.ShapeDtypeStruct((2 * npad, hc), jnp.float32),
                   jax.ShapeDtypeStruct((2 * npad, hc), jnp.float32)],
    )

    def round_body(k, carry):
        y, g = carry
        p = edge_call(y, row3, row3b, col3, zblk)
        gk = lax.dynamic_slice(gamma, (k,), (1,))
        return combine_call(gk, p, p, dinv16, g)

    y, g = lax.fori_loop(1, kk + 1, round_body, (y2_0, g2_0))

    final_call = pl.pallas_call(
        functools.partial(_final_body, hc),
        grid=(grid,),
        in_specs=[pl.BlockSpec((rt, 16), lambda i: (i, 0)),
                  pl.BlockSpec((rt, hc), lambda i: (i, 0)),
                  pl.BlockSpec((rt, hc), lambda i: (i + grid, 0))],
        out_specs=pl.BlockSpec((rt, out_ch), lambda i: (i, 0)),
        out_shape=jax.ShapeDtypeStruct((npad, out_ch), jnp.float32),
    )
    hidden = final_call(dsq16, g, g)
    return hidden[:n]


# speedup vs baseline: 9.1819x; 1.8027x over previous
import functools

import jax
import jax.numpy as jnp
from jax import lax
from jax.experimental import pallas as pl
from jax.experimental.pallas import tpu as pltpu
from jax.experimental.pallas import tpu_sc as plsc

BN_EPS = 1e-5
NC = 2
NS = 16
NW = NC * NS
B = 128


NBUF = 6


def _edge_body(n_chunks, npad, y2, rowg, cols, zblk, p_out,
               rowi, coli, buf, gsems, ssems, acc):
    cid = lax.axis_index("c")
    sid = lax.axis_index("s")
    wid = cid * NS + sid
    rt = npad // NS

    pltpu.sync_copy(rowg.at[wid], rowi)
    pltpu.sync_copy(cols.at[wid], coli)

    plsc.subcore_barrier()
    pltpu.sync_copy(zblk, acc.at[pl.ds(sid * rt, rt)])
    plsc.subcore_barrier()

    gd = {}
    sd = {}
    for b in range(min(3, n_chunks)):
        gd[b] = pltpu.async_copy(y2.at[rowi.at[b]], buf.at[b], gsems.at[b])
    for j in range(n_chunks):
        b = j % NBUF
        pj = j + 3
        if pj < n_chunks:
            pb = pj % NBUF
            if pb in sd:
                sd.pop(pb).wait()
            gd[pb] = pltpu.async_copy(
                y2.at[rowi.at[pj]], buf.at[pb], gsems.at[pb])
        gd.pop(b).wait()
        sd[b] = pltpu.async_copy(
            buf.at[b], acc.at[coli.at[j]], ssems.at[b], add=True)
    for b in list(sd):
        sd.pop(b).wait()

    plsc.subcore_barrier()
    pltpu.sync_copy(
        acc.at[pl.ds(sid * rt, rt)],
        p_out.at[pl.ds(cid * npad + sid * rt, rt)])


def _mlp_body(gamma_ref, x_ref, w1_ref, b1_ref, bnw_ref, bnb_ref, w2_ref,
              b2_ref, degp_ref, y0_ref, g0_ref, dinv16_ref, dsq16_ref):
    h = lax.dot_general(x_ref[...], w1_ref[...], (((1,), (1,)), ((), ())),
                        preferred_element_type=jnp.float32)
    scale = bnw_ref[...] * lax.rsqrt(jnp.float32(1.0 + BN_EPS))
    h = h * scale + b1_ref[...] * scale + bnb_ref[...]
    h = jnp.maximum(h, 0.0)
    h = lax.dot_general(h, w2_ref[...], (((1,), (1,)), ((), ())),
                        preferred_element_type=jnp.float32) + b2_ref[...]
    d = jnp.maximum(degp_ref[:, 0:1] + degp_ref[:, 1:2], 1.0)
    dis = lax.rsqrt(d)
    y0 = h * dis
    y0_ref[...] = y0
    g0_ref[...] = y0 * gamma_ref[0]
    ones16 = jnp.ones((1, 16), jnp.float32)
    dinv16_ref[...] = (dis * dis) * ones16
    dsq16_ref[...] = (d * dis) * ones16


def _combine_body(gk_ref, p_ref, dinv16_ref, g_ref, y_ref, gout_ref):
    y = p_ref[...] * dinv16_ref[:, 0:1]
    y_ref[...] = y
    gout_ref[...] = g_ref[...] + y * gk_ref[0]


def _final_body(hc, dsq16_ref, ga_ref, gb_ref, out_ref):
    d = dsq16_ref[:, 0:1]
    out_ref[:, :hc] = ga_ref[...] * d
    out_ref[:, hc:] = gb_ref[...] * d


def kernel(x, edge_index, W1, b1, bn_w, bn_b, W2, b2, gamma):
    n, in_ch = x.shape
    e = edge_index.shape[1]
    kk = gamma.shape[0] - 1
    out_ch = W2.shape[0]

    n_chunks = -(-e // (NS * B))
    e_pad = NS * B * n_chunks
    npad = -(-(n + 1) // 256) * 256
    rt = npad // NS

    row = edge_index[0]
    col = edge_index[1]
    pad_idx = jnp.full((e_pad - e,), n, jnp.int32)
    rowr = jnp.concatenate([row, pad_idx]).reshape(NS, n_chunks, B)
    colr = jnp.concatenate([col, pad_idx]).reshape(NS, n_chunks, B)
    rowg = jnp.concatenate([rowr, rowr + npad], axis=0)
    cols = jnp.concatenate([colr, colr], axis=0)
    x_p = jnp.pad(x, ((0, npad - n), (0, 0)))
    zblk = jnp.zeros((rt, out_ch // 2), jnp.float32)

    mesh = plsc.VectorSubcoreMesh(core_axis_name="c", subcore_axis_name="s",
                                  num_cores=NC, num_subcores=NS)
    sc_params = pltpu.CompilerParams(needs_layout_passes=False,
                                     use_tc_tiling_on_sc=False)

    hc = out_ch // 2
    edge_call = pl.kernel(
        functools.partial(_edge_body, n_chunks, npad),
        out_type=jax.ShapeDtypeStruct((NC * npad, hc), jnp.float32),
        mesh=mesh,
        scratch_types=[
            pltpu.VMEM((n_chunks, B), jnp.int32),
            pltpu.VMEM((n_chunks, B), jnp.int32),
            pltpu.VMEM((NBUF, B, hc), jnp.float32),
            pltpu.SemaphoreType.DMA((NBUF,)),
            pltpu.SemaphoreType.DMA((NBUF,)),
            pltpu.VMEM_SHARED((npad, hc), jnp.float32),
        ],
        compiler_params=sc_params,
    )

    halves = jnp.full((2 * npad, hc), 0.5, jnp.float32)
    rows_s = jnp.concatenate([rowr, rowr], axis=0)
    p_deg = edge_call(halves, rowg, rows_s, zblk)
    degp = jnp.stack([p_deg[:npad, 0], p_deg[npad:2 * npad, 0]], axis=1)

    grid = npad // rt

    mlp_call = pl.pallas_call(
        _mlp_body,
        grid=(grid,),
        in_specs=[
            pl.BlockSpec(memory_space=pltpu.SMEM),
            pl.BlockSpec((rt, in_ch), lambda i: (i, 0)),
            pl.BlockSpec(W1.shape, lambda i: (0, 0)),
            pl.BlockSpec((1, in_ch), lambda i: (0, 0)),
            pl.BlockSpec((1, in_ch), lambda i: (0, 0)),
            pl.BlockSpec((1, in_ch), lambda i: (0, 0)),
            pl.BlockSpec(W2.shape, lambda i: (0, 0)),
            pl.BlockSpec((1, out_ch), lambda i: (0, 0)),
            pl.BlockSpec((rt, NC), lambda i: (i, 0)),
        ],
        out_specs=[
            pl.BlockSpec((rt, out_ch), lambda i: (i, 0)),
            pl.BlockSpec((rt, out_ch), lambda i: (i, 0)),
            pl.BlockSpec((rt, 16), lambda i: (i, 0)),
            pl.BlockSpec((rt, 16), lambda i: (i, 0)),
        ],
        out_shape=[
            jax.ShapeDtypeStruct((npad, out_ch), jnp.float32),
            jax.ShapeDtypeStruct((npad, out_ch), jnp.float32),
            jax.ShapeDtypeStruct((npad, 16), jnp.float32),
            jax.ShapeDtypeStruct((npad, 16), jnp.float32),
        ],
    )
    y0, g0, dinv16, dsq16 = mlp_call(
        gamma, x_p, W1, b1.reshape(1, -1), bn_w.reshape(1, -1),
        bn_b.reshape(1, -1), W2, b2.reshape(1, -1), degp)

    y2_0 = jnp.concatenate([y0[:, :hc], y0[:, hc:]], axis=0)
    g2_0 = jnp.concatenate([g0[:, :hc], g0[:, hc:]], axis=0)

    grid2 = 2 * grid
    combine_call = pl.pallas_call(
        _combine_body,
        grid=(grid2,),
        in_specs=[
            pl.BlockSpec(memory_space=pltpu.SMEM),
            pl.BlockSpec((rt, hc), lambda i: (i, 0)),
            pl.BlockSpec((rt, 16), lambda i: (i % grid, 0)),
            pl.BlockSpec((rt, hc), lambda i: (i, 0)),
        ],
        out_specs=[pl.BlockSpec((rt, hc), lambda i: (i, 0)),
                   pl.BlockSpec((rt, hc), lambda i: (i, 0))],
        out_shape=[jax.ShapeDtypeStruct((2 * npad, hc), jnp.float32),
                   jax.ShapeDtypeStruct((2 * npad, hc), jnp.float32)],
    )

    def round_body(k, carry):
        y, g = carry
        p = edge_call(y, rowg, cols, zblk)
        gk = lax.dynamic_slice(gamma, (k,), (1,))
        return combine_call(gk, p, dinv16, g)

    y, g = lax.fori_loop(1, kk + 1, round_body, (y2_0, g2_0))

    final_call = pl.pallas_call(
        functools.partial(_final_body, hc),
        grid=(grid,),
        in_specs=[pl.BlockSpec((rt, 16), lambda i: (i, 0)),
                  pl.BlockSpec((rt, hc), lambda i: (i, 0)),
                  pl.BlockSpec((rt, hc), lambda i: (i + grid, 0))],
        out_specs=pl.BlockSpec((rt, out_ch), lambda i: (i, 0)),
        out_shape=jax.ShapeDtypeStruct((npad, out_ch), jnp.float32),
    )
    hidden = final_call(dsq16, g, g)
    return hidden[:n]
